# baseline (speedup 1.0000x reference)
import jax
import jax.numpy as jnp
from jax.experimental import pallas as pl
from jax.experimental.pallas import tpu as pltpu


def _make_kernel(ct, n_ct, inv_s):
    def _gate_kernel(x_ref, w1_ref, b1_ref, w2_ref, b2_ref, o_ref,
                     sum_scr, max_scr):
        tc = pl.program_id(1)
        x = x_ref[...]
        o_ref[pl.ds(tc * ct, ct), :] = x
        sum_scr[pl.ds(tc * ct, ct), :] = jnp.sum(x, axis=-1, keepdims=True)
        max_scr[pl.ds(tc * ct, ct), :] = jnp.max(x, axis=-1, keepdims=True)

        @pl.when(tc == n_ct - 1)
        def _():
            p = jnp.concatenate(
                [sum_scr[...] * inv_s, max_scr[...]], axis=1)
            h = jnp.dot(w1_ref[...], p,
                        preferred_element_type=jnp.float32) + b1_ref[...]
            h = jnp.maximum(h, 0.0)
            a = jnp.dot(w2_ref[...], h,
                        preferred_element_type=jnp.float32) + b2_ref[...]
            att = a[:, :1] + a[:, 1:2]
            scale = jax.nn.sigmoid(att)
            o_ref[...] = (o_ref[...] * scale).astype(o_ref.dtype)

    return _gate_kernel


def kernel(x, w1, b1, w2, b2):
    N, C, D, H, W = x.shape
    S = D * H * W
    Ch = w1.shape[0]

    w1f = jnp.asarray(w1, jnp.float32)
    w2f = jnp.asarray(w2, jnp.float32)
    b1r = jnp.asarray(b1, jnp.float32).reshape(Ch, 1)
    b2r = jnp.asarray(b2, jnp.float32).reshape(C, 1)

    x2 = x.reshape(N * C, S)

    ct = 32
    while C % ct != 0:
        ct //= 2
    n_ct = C // ct

    item = jnp.dtype(x.dtype).itemsize
    blk_bytes = C * S * item
    limit = min(2 * blk_bytes + 4 * ct * S * item + (4 << 20),
                60 * 1024 * 1024)

    out2 = pl.pallas_call(
        _make_kernel(ct, n_ct, 1.0 / S),
        out_shape=jax.ShapeDtypeStruct((N * C, S), x.dtype),
        grid=(N, n_ct),
        in_specs=[
            pl.BlockSpec((ct, S), lambda n, t: (n * n_ct + t, 0)),
            pl.BlockSpec((Ch, C), lambda n, t: (0, 0)),
            pl.BlockSpec((Ch, 1), lambda n, t: (0, 0)),
            pl.BlockSpec((C, Ch), lambda n, t: (0, 0)),
            pl.BlockSpec((C, 1), lambda n, t: (0, 0)),
        ],
        out_specs=pl.BlockSpec((C, S), lambda n, t: (n, 0)),
        scratch_shapes=[
            pltpu.VMEM((C, 1), jnp.float32),
            pltpu.VMEM((C, 1), jnp.float32),
        ],
        compiler_params=pltpu.CompilerParams(
            dimension_semantics=("arbitrary", "arbitrary"),
            vmem_limit_bytes=int(limit),
        ),
    )(x2, w1f, b1r, w2f, b2r)
    return out2.reshape(N, C, D, H, W)


# speedup vs baseline: 1.0430x; 1.0430x over previous
import jax
import jax.numpy as jnp
from jax.experimental import pallas as pl
from jax.experimental.pallas import tpu as pltpu


def _make_kernel(n_items, n_tiles, st, inv_s):
    def _body(x_ref, w1_ref, b1_ref, w2_ref, b2_ref, o_ref,
              xbuf, sum_scr, max_scr, scale_scr):
        n = pl.program_id(0)
        t = pl.program_id(1)
        p = jax.lax.rem(n, 2)
        q = 1 - p

        @pl.when(n < n_items)
        def _():
            xt = x_ref[...]
            xbuf[p, :, pl.ds(t * st, st)] = xt
            s_part = jnp.sum(xt, axis=-1, keepdims=True)
            m_part = jnp.max(xt, axis=-1, keepdims=True)

            @pl.when(t == 0)
            def _():
                sum_scr[p] = s_part
                max_scr[p] = m_part

            @pl.when(t > 0)
            def _():
                sum_scr[p] += s_part
                max_scr[p] = jnp.maximum(max_scr[p], m_part)

        @pl.when((n > 0) & (t == 0))
        def _():
            pstat = jnp.concatenate(
                [sum_scr[q] * inv_s, max_scr[q]], axis=1)
            h = jnp.dot(w1_ref[...], pstat,
                        preferred_element_type=jnp.float32) + b1_ref[...]
            h = jnp.maximum(h, 0.0)
            a = jnp.dot(w2_ref[...], h,
                        preferred_element_type=jnp.float32) + b2_ref[...]
            att = a[:, :1] + a[:, 1:2]
            scale_scr[q] = jax.nn.sigmoid(att)

        @pl.when(n > 0)
        def _():
            o_ref[...] = (xbuf[q, :, pl.ds(t * st, st)]
                          * scale_scr[q]).astype(o_ref.dtype)

    return _body


def kernel(x, w1, b1, w2, b2):
    N, C, D, H, W = x.shape
    S = D * H * W
    Ch = w1.shape[0]

    w1f = jnp.asarray(w1, jnp.float32)
    w2f = jnp.asarray(w2, jnp.float32)
    b1r = jnp.asarray(b1, jnp.float32).reshape(Ch, 1)
    b2r = jnp.asarray(b2, jnp.float32).reshape(C, 1)

    x2 = x.reshape(N * C, S)

    st = S
    for cand in (2048, 1024, 512, 256, 128):
        if S % cand == 0:
            st = cand
            break
    n_tiles = S // st

    item = jnp.dtype(x.dtype).itemsize
    slab = C * S * item
    tile = C * st * item
    limit = min(2 * slab + 4 * tile + (4 << 20), 60 * 1024 * 1024)

    in_map = lambda n, t: (jnp.minimum(n, N - 1),
                           jnp.where(n < N, t, 0))
    out_map = lambda n, t: (jnp.where(n > 0, n - 1, N - 1), t)

    out2 = pl.pallas_call(
        _make_kernel(N, n_tiles, st, 1.0 / S),
        out_shape=jax.ShapeDtypeStruct((N * C, S), x.dtype),
        grid=(N + 1, n_tiles),
        in_specs=[
            pl.BlockSpec((C, st), in_map),
            pl.BlockSpec((Ch, C), lambda n, t: (0, 0)),
            pl.BlockSpec((Ch, 1), lambda n, t: (0, 0)),
            pl.BlockSpec((C, Ch), lambda n, t: (0, 0)),
            pl.BlockSpec((C, 1), lambda n, t: (0, 0)),
        ],
        out_specs=pl.BlockSpec((C, st), out_map),
        scratch_shapes=[
            pltpu.VMEM((2, C, S), jnp.float32),
            pltpu.VMEM((2, C, 1), jnp.float32),
            pltpu.VMEM((2, C, 1), jnp.float32),
            pltpu.VMEM((2, C, 1), jnp.float32),
        ],
        compiler_params=pltpu.CompilerParams(
            dimension_semantics=("arbitrary", "arbitrary"),
            vmem_limit_bytes=int(limit),
        ),
    )(x2, w1f, b1r, w2f, b2r)
    return out2.reshape(N, C, D, H, W)


# speedup vs baseline: 1.0843x; 1.0396x over previous
import jax
import jax.numpy as jnp
from jax.experimental import pallas as pl
from jax.experimental.pallas import tpu as pltpu


def _make_body(n_items, c, inv_s):
    def _body(x_hbm, w1_ref, b1_ref, w2_ref, b2_ref, o_hbm,
              slab, sem_in, sem_out):
        n = pl.program_id(0)
        p = jax.lax.rem(n, 2)
        q = 1 - p

        @pl.when(n < n_items)
        def _():
            @pl.when(n >= 2)
            def _():
                pltpu.make_async_copy(
                    slab.at[p], slab.at[p], sem_out.at[p]).wait()
            pltpu.make_async_copy(
                x_hbm.at[pl.ds(n * c, c), :], slab.at[p],
                sem_in.at[p]).start()

        @pl.when(n >= 1)
        def _():
            pltpu.make_async_copy(
                slab.at[q], slab.at[q], sem_in.at[q]).wait()
            xq = slab[q]
            ssum = jnp.sum(xq, axis=-1, keepdims=True)
            smax = jnp.max(xq, axis=-1, keepdims=True)
            pstat = jnp.concatenate([ssum * inv_s, smax], axis=1)
            h = jnp.dot(w1_ref[...], pstat,
                        preferred_element_type=jnp.float32) + b1_ref[...]
            h = jnp.maximum(h, 0.0)
            a = jnp.dot(w2_ref[...], h,
                        preferred_element_type=jnp.float32) + b2_ref[...]
            scale = jax.nn.sigmoid(a[:, :1] + a[:, 1:2])
            slab[q] = xq * scale
            pltpu.make_async_copy(
                slab.at[q], o_hbm.at[pl.ds((n - 1) * c, c), :],
                sem_out.at[q]).start()

        @pl.when(n == n_items)
        def _():
            if n_items >= 2:
                pltpu.make_async_copy(
                    slab.at[p], slab.at[p], sem_out.at[p]).wait()
            pltpu.make_async_copy(
                slab.at[q], slab.at[q], sem_out.at[q]).wait()

    return _body


def kernel(x, w1, b1, w2, b2):
    N, C, D, H, W = x.shape
    S = D * H * W
    Ch = w1.shape[0]

    w1f = jnp.asarray(w1, jnp.float32)
    w2f = jnp.asarray(w2, jnp.float32)
    b1r = jnp.asarray(b1, jnp.float32).reshape(Ch, 1)
    b2r = jnp.asarray(b2, jnp.float32).reshape(C, 1)

    x2 = x.reshape(N * C, S)

    item = jnp.dtype(x.dtype).itemsize
    slab_bytes = 2 * C * S * item
    limit = min(slab_bytes + (4 << 20), 60 * 1024 * 1024)

    out2 = pl.pallas_call(
        _make_body(N, C, 1.0 / S),
        out_shape=jax.ShapeDtypeStruct((N * C, S), x.dtype),
        grid=(N + 1,),
        in_specs=[
            pl.BlockSpec(memory_space=pl.ANY),
            pl.BlockSpec((Ch, C), lambda n: (0, 0)),
            pl.BlockSpec((Ch, 1), lambda n: (0, 0)),
            pl.BlockSpec((C, Ch), lambda n: (0, 0)),
            pl.BlockSpec((C, 1), lambda n: (0, 0)),
        ],
        out_specs=pl.BlockSpec(memory_space=pl.ANY),
        scratch_shapes=[
            pltpu.VMEM((2, C, S), jnp.float32),
            pltpu.SemaphoreType.DMA((2,)),
            pltpu.SemaphoreType.DMA((2,)),
        ],
        compiler_params=pltpu.CompilerParams(
            dimension_semantics=("arbitrary",),
            vmem_limit_bytes=int(limit),
        ),
    )(x2, w1f, b1r, w2f, b2r)
    return out2.reshape(N, C, D, H, W)
